# baseline (speedup 1.0000x reference)
import functools

import jax
import jax.numpy as jnp
from jax import lax
from jax.experimental import pallas as pl
from jax.experimental.pallas import tpu as pltpu
from jax.experimental.pallas import tpu_sc as plsc

_NS = 65536
_NG = 512
_NE = 320000
_D = 128

_NC = 2
_NSUB = 16
_NW = _NC * _NSUB

_A_SEG_ROWS = 248
_A_SEG_XTRA = 8
_A_GID_ROWS = _NS // 64 // _NSUB
_A_SEG_PER_TILE = _NS // _NC // _NSUB

_B_CHUNK = 80
_B_NCHUNK = _NE // _NW // _B_CHUNK


def _mesh():
    return plsc.VectorSubcoreMesh(core_axis_name="c", subcore_axis_name="s")


@functools.partial(
    pl.kernel,
    out_type=(jax.ShapeDtypeStruct((_NS,), jnp.float32),
              jax.ShapeDtypeStruct((_NG,), jnp.float32)),
    mesh=_mesh(),
    scratch_types=[
        pltpu.VMEM((_A_SEG_ROWS, 80), jnp.int32),
        pltpu.VMEM((_A_SEG_XTRA, 80), jnp.int32),
        pltpu.VMEM((_A_GID_ROWS, 64), jnp.int32),
        pltpu.VMEM((4096,), jnp.float32),
        pltpu.VMEM((80,), jnp.float32),
        pltpu.VMEM((_A_SEG_PER_TILE,), jnp.float32),
        pltpu.VMEM((_NG,), jnp.float32),
        pltpu.VMEM((_A_SEG_PER_TILE,), jnp.float32),
        pltpu.VMEM_SHARED((_NS,), jnp.float32),
        pltpu.VMEM_SHARED((_NG,), jnp.float32),
        pltpu.SemaphoreType.DMA,
    ],
)
def _pass_a(seg2d, gids2d, q_out, gc_out, ids_v, ids_x, gids_v, zeros_v, ones_v,
            cnt_v, gc_v, qv, counts_sh, gc_sh, sem_a):
    c = lax.axis_index("c")
    s = lax.axis_index("s")

    def fill_z(i, _):
        zeros_v[pl.ds(16 * i, 16)] = jnp.zeros((16,), jnp.float32)
        return 0
    lax.fori_loop(0, 4096 // 16, fill_z, 0)

    def fill_o(i, _):
        ones_v[pl.ds(16 * i, 16)] = jnp.ones((16,), jnp.float32)
        return 0
    lax.fori_loop(0, 80 // 16, fill_o, 0)

    pltpu.sync_copy(zeros_v, counts_sh.at[pl.ds(4096 * s, 4096)])

    @pl.when(s == 0)
    def _():
        pltpu.sync_copy(zeros_v.at[pl.ds(0, _NG)], gc_sh)

    plsc.subcore_barrier()

    pltpu.sync_copy(seg2d.at[pl.ds(_A_SEG_ROWS * s, _A_SEG_ROWS)], ids_v)
    pltpu.sync_copy(gids2d.at[pl.ds(_A_GID_ROWS * s, _A_GID_ROWS)], gids_v)

    @pl.when(s < 4)
    def _():
        pltpu.sync_copy(
            seg2d.at[pl.ds(_A_SEG_ROWS * _NSUB + _A_SEG_XTRA * s, _A_SEG_XTRA)],
            ids_x)

    def cnt_fire(j, _):
        pltpu.async_copy(ones_v, counts_sh.at[ids_v.at[j]], sem_a, add=True)
        return 0
    lax.fori_loop(0, _A_SEG_ROWS, cnt_fire, 0)

    @pl.when(s < 4)
    def _():
        def cnt_tail(j, _):
            pltpu.async_copy(ones_v, counts_sh.at[ids_x.at[j]], sem_a, add=True)
            return 0
        lax.fori_loop(0, _A_SEG_XTRA, cnt_tail, 0)

    def gc_fire(j, _):
        pltpu.async_copy(ones_v.at[pl.ds(0, 64)], gc_sh.at[gids_v.at[j]],
                         sem_a, add=True)
        return 0
    lax.fori_loop(0, _A_GID_ROWS, gc_fire, 0)

    def cnt_drain(j, _):
        pltpu.make_async_copy(ones_v, counts_sh.at[ids_v.at[0]], sem_a).wait()
        return 0
    lax.fori_loop(0, _A_SEG_ROWS, cnt_drain, 0)

    @pl.when(s < 4)
    def _():
        def tail_drain(j, _):
            pltpu.make_async_copy(ones_v, counts_sh.at[ids_x.at[0]],
                                  sem_a).wait()
            return 0
        lax.fori_loop(0, _A_SEG_XTRA, tail_drain, 0)

    def gc_drain(j, _):
        pltpu.make_async_copy(ones_v.at[pl.ds(0, 64)], gc_sh.at[gids_v.at[0]],
                              sem_a).wait()
        return 0
    lax.fori_loop(0, _A_GID_ROWS, gc_drain, 0)

    plsc.subcore_barrier()

    base_s = (_NS // _NC) * c + _A_SEG_PER_TILE * s
    pltpu.sync_copy(counts_sh.at[pl.ds(base_s, _A_SEG_PER_TILE)], cnt_v)

    def q_body(i, _):
        sl = pl.ds(16 * i, 16)
        qv[sl] = 1.0 / jnp.maximum(cnt_v[sl], 1.0)
        return 0
    lax.fori_loop(0, _A_SEG_PER_TILE // 16, q_body, 0)

    pltpu.sync_copy(qv, q_out.at[pl.ds(base_s, _A_SEG_PER_TILE)])

    @pl.when(jnp.logical_and(c == 0, s == 0))
    def _():
        pltpu.sync_copy(gc_sh, gc_v)
        pltpu.sync_copy(gc_v, gc_out)


@functools.partial(
    pl.kernel,
    out_type=jax.ShapeDtypeStruct((_NW, _NG, _D), jnp.float32),
    mesh=_mesh(),
    scratch_types=[
        pltpu.VMEM((_B_NCHUNK * _B_CHUNK,), jnp.int32),
        pltpu.VMEM((_B_CHUNK,), jnp.float32),
        pltpu.VMEM((_B_CHUNK,), jnp.float32),
        pltpu.VMEM((_B_CHUNK,), jnp.float32),
        pltpu.VMEM((_B_CHUNK,), jnp.float32),
        pltpu.VMEM((_B_CHUNK,), jnp.int32),
        pltpu.VMEM((_B_CHUNK,), jnp.int32),
        pltpu.VMEM((_B_CHUNK,), jnp.int32),
        pltpu.VMEM((_B_CHUNK,), jnp.int32),
        pltpu.VMEM((_B_CHUNK, _D), jnp.float32),
        pltpu.VMEM((_B_CHUNK, _D), jnp.float32),
        pltpu.VMEM((_B_CHUNK, _D), jnp.float32),
        pltpu.VMEM((_B_CHUNK, _D), jnp.float32),
        pltpu.VMEM((_NG, _D), jnp.float32),
        pltpu.SemaphoreType.DMA,
        pltpu.SemaphoreType.DMA,
        pltpu.SemaphoreType.DMA,
        pltpu.SemaphoreType.DMA,
        pltpu.SemaphoreType.DMA,
        pltpu.SemaphoreType.DMA,
        pltpu.SemaphoreType.DMA,
        pltpu.SemaphoreType.DMA,
        pltpu.SemaphoreType.DMA,
        pltpu.SemaphoreType.DMA,
        pltpu.SemaphoreType.DMA,
        pltpu.SemaphoreType.DMA,
    ],
    compiler_params=pltpu.CompilerParams(needs_layout_passes=False),
)
def _pass_b(edge_feat, seg1d, g1d, q_hbm, out, ids_v,
            w_v0, w_v1, w_v2, w_v3, g_v0, g_v1, g_v2, g_v3,
            buf0, buf1, buf2, buf3, acc,
            sem_w0, sem_w1, sem_w2, sem_w3,
            sem_g0, sem_g1, sem_g2, sem_g3,
            sem_x0, sem_x1, sem_x2, sem_x3):
    c = lax.axis_index("c")
    s = lax.axis_index("s")
    wid = s * _NC + c
    n_tile = _B_NCHUNK * _B_CHUNK

    slots = ((w_v0, g_v0, buf0, sem_w0, sem_g0, sem_x0),
             (w_v1, g_v1, buf1, sem_w1, sem_g1, sem_x1),
             (w_v2, g_v2, buf2, sem_w2, sem_g2, sem_x2),
             (w_v3, g_v3, buf3, sem_w3, sem_g3, sem_x3))

    pltpu.sync_copy(seg1d.at[pl.ds(n_tile * wid, n_tile)], ids_v)

    def _start(j, slot):
        w_v, g_v, buf, sem_w, sem_g, sem_x = slot
        idx_row = ids_v.at[pl.ds(_B_CHUNK * j, _B_CHUNK)]
        pltpu.async_copy(q_hbm.at[idx_row], w_v, sem_w)
        pltpu.async_copy(g1d.at[idx_row], g_v, sem_g)
        base_e = _B_CHUNK * (_B_NCHUNK * wid + j)
        pltpu.async_copy(edge_feat.at[pl.ds(base_e, _B_CHUNK)], buf, sem_x)

    def _wait(slot):
        w_v, g_v, buf, sem_w, sem_g, sem_x = slot
        pltpu.make_async_copy(q_hbm.at[pl.ds(0, _B_CHUNK)], w_v, sem_w).wait()
        pltpu.make_async_copy(g1d.at[pl.ds(0, _B_CHUNK)], g_v, sem_g).wait()
        pltpu.make_async_copy(edge_feat.at[pl.ds(0, _B_CHUNK)], buf, sem_x).wait()

    iota16 = lax.iota(jnp.int32, 16)
    _gdn = lax.GatherDimensionNumbers(
        offset_dims=(), collapsed_slice_dims=(0,), start_index_map=(0,))

    def _lane_bcast(vec, lane):
        idx = jnp.full((16, 1), lane, jnp.int32)
        return lax.gather(vec, idx, _gdn, slice_sizes=(1,),
                          mode=lax.GatherScatterMode.PROMISE_IN_BOUNDS)

    def _flush(R, gcur):
        for col in range(_D // 16):
            plsc.addupdate_scatter(acc, [gcur, iota16 + 16 * col], R[col])

    def _accum_group(k, wv, buf, R):
        wsps = [_lane_bcast(wv, lane) for lane in range(16)]
        Rl = list(R)
        for col in range(_D // 16):
            sl = pl.ds(16 * col, 16)
            prods = [buf[16 * k + lane, sl] * wsps[lane] for lane in range(16)]
            while len(prods) > 1:
                prods = [prods[p] + prods[p + 1]
                         for p in range(0, len(prods), 2)]
            Rl[col] = Rl[col] + prods[0]
        return tuple(Rl)

    def _compute(slot, carry):
        w_v, g_v, buf, _, _, _ = slot
        R0, gcur0 = carry
        gvs = [g_v[pl.ds(16 * k, 16)] for k in range(_B_CHUNK // 16)]
        u = jnp.all(gvs[0] == gcur0)
        for k in range(1, _B_CHUNK // 16):
            u = jnp.logical_and(u, jnp.all(gvs[k] == gcur0))

        def fast(R, gcur):
            def gbody(k, Rt):
                wv = w_v[pl.ds(16 * k, 16)]
                return _accum_group(k, wv, buf, Rt)
            return lax.fori_loop(0, _B_CHUNK // 16, gbody, R), gcur

        def slow(R, gcur):
            def gbody(k, car):
                Rk, gk = car
                wv = w_v[pl.ds(16 * k, 16)]
                gv = g_v[pl.ds(16 * k, 16)]
                uniform = jnp.all(gv == gk)

                def gfast(Rx, gx):
                    def lane_body(lane, Rt):
                        wsp = _lane_bcast(wv, lane)
                        i = 16 * k + lane
                        return tuple(
                            Rt[col] + buf[i, pl.ds(16 * col, 16)] * wsp
                            for col in range(_D // 16))
                    return lax.fori_loop(0, 16, lane_body, Rx), gx

                def gslow(Rx, gx):
                    _flush(Rx, gx)

                    def lane_body(lane, _):
                        wsp = _lane_bcast(wv, lane)
                        gsp = _lane_bcast(gv, lane)
                        i = 16 * k + lane
                        for col in range(_D // 16):
                            plsc.addupdate_scatter(
                                acc, [gsp, iota16 + 16 * col],
                                buf[i, pl.ds(16 * col, 16)] * wsp)
                        return 0
                    lax.fori_loop(0, 16, lane_body, 0)
                    z = jnp.zeros((16,), jnp.float32)
                    return (tuple(z for _ in range(_D // 16)),
                            _lane_bcast(gv, 15))

                return lax.cond(uniform, gfast, gslow, Rk, gk)
            return lax.fori_loop(0, _B_CHUNK // 16, gbody, (R, gcur))

        return lax.cond(u, fast, slow, R0, gcur0)

    _start(0, slots[0])
    _start(1, slots[1])
    _start(2, slots[2])

    def z_body(r, _):
        for col in range(_D // 16):
            acc[r, pl.ds(16 * col, 16)] = jnp.zeros((16,), jnp.float32)
        return 0
    lax.fori_loop(0, _NG, z_body, 0)

    carry0 = (tuple(jnp.zeros((16,), jnp.float32) for _ in range(_D // 16)),
              jnp.zeros((16,), jnp.int32))

    def quad_body(t, carry):
        j0 = 4 * t
        for u in range(4):
            j = j0 + u

            @pl.when(j + 3 < _B_NCHUNK)
            def _():
                _start(j + 3, slots[(u + 3) % 4])

            def do(R, gcur):
                _wait(slots[u])
                return _compute(slots[u], (R, gcur))

            def skip(R, gcur):
                return (R, gcur)

            carry = lax.cond(j < _B_NCHUNK, do, skip, *carry)
        return carry
    carry = lax.fori_loop(0, (_B_NCHUNK + 3) // 4, quad_body, carry0)

    R, gcur = carry
    _flush(R, gcur)

    pltpu.sync_copy(acc, out.at[wid])


def _sum_body(x_ref, gc_ref, o_ref):
    tot = jnp.sum(x_ref[...], axis=0)
    gc = jnp.maximum(gc_ref[...], 1.0)
    o_ref[...] = tot / gc[:, None]


def kernel(edge_feat, segment_ids, graph_ids):
    seg_i = segment_ids.astype(jnp.int32)
    g_i = graph_ids.astype(jnp.int32)
    seg2d = seg_i.reshape(_NE // 80, 80)
    gids2d = g_i.reshape(_NS // 64, 64)

    q, gc = _pass_a(seg2d, gids2d)
    part = _pass_b(edge_feat, seg_i, g_i, q)

    pooled = pl.pallas_call(
        _sum_body,
        out_shape=jax.ShapeDtypeStruct((_NG, _D), jnp.float32),
    )(part, gc)
    return pooled


# speedup vs baseline: 1.2389x; 1.2389x over previous
import functools

import jax
import jax.numpy as jnp
from jax import lax
from jax.experimental import pallas as pl
from jax.experimental.pallas import tpu as pltpu
from jax.experimental.pallas import tpu_sc as plsc

_NS = 65536
_NG = 512
_NE = 320000
_D = 128

_NC = 2
_NSUB = 16
_NW = _NC * _NSUB

_A_SEG_ROWS = 248
_A_SEG_XTRA = 8
_A_GID_ROWS = _NS // 64 // _NSUB
_A_SEG_PER_TILE = _NS // _NC // _NSUB

_B_CHUNK = 80
_B_NCHUNK = _NE // _NW // _B_CHUNK


def _mesh():
    return plsc.VectorSubcoreMesh(core_axis_name="c", subcore_axis_name="s")


@functools.partial(
    pl.kernel,
    out_type=(jax.ShapeDtypeStruct((_NS,), jnp.float32),
              jax.ShapeDtypeStruct((_NG,), jnp.float32)),
    mesh=_mesh(),
    scratch_types=[
        pltpu.VMEM((_A_SEG_ROWS, 80), jnp.int32),
        pltpu.VMEM((_A_SEG_XTRA, 80), jnp.int32),
        pltpu.VMEM((_A_GID_ROWS, 64), jnp.int32),
        pltpu.VMEM((4096,), jnp.float32),
        pltpu.VMEM((80,), jnp.float32),
        pltpu.VMEM((_A_SEG_PER_TILE,), jnp.float32),
        pltpu.VMEM((_NG,), jnp.float32),
        pltpu.VMEM((_A_SEG_PER_TILE,), jnp.float32),
        pltpu.VMEM_SHARED((_NS,), jnp.float32),
        pltpu.VMEM_SHARED((_NG,), jnp.float32),
        pltpu.SemaphoreType.DMA,
    ],
)
def _pass_a(seg2d, gids2d, q_out, gc_out, ids_v, ids_x, gids_v, zeros_v, ones_v,
            cnt_v, gc_v, qv, counts_sh, gc_sh, sem_a):
    c = lax.axis_index("c")
    s = lax.axis_index("s")

    def fill_z(i, _):
        zeros_v[pl.ds(16 * i, 16)] = jnp.zeros((16,), jnp.float32)
        return 0
    lax.fori_loop(0, 4096 // 16, fill_z, 0)

    def fill_o(i, _):
        ones_v[pl.ds(16 * i, 16)] = jnp.ones((16,), jnp.float32)
        return 0
    lax.fori_loop(0, 80 // 16, fill_o, 0)

    pltpu.sync_copy(zeros_v, counts_sh.at[pl.ds(4096 * s, 4096)])

    @pl.when(s == 0)
    def _():
        pltpu.sync_copy(zeros_v.at[pl.ds(0, _NG)], gc_sh)

    plsc.subcore_barrier()

    pltpu.sync_copy(seg2d.at[pl.ds(_A_SEG_ROWS * s, _A_SEG_ROWS)], ids_v)
    pltpu.sync_copy(gids2d.at[pl.ds(_A_GID_ROWS * s, _A_GID_ROWS)], gids_v)

    @pl.when(s < 4)
    def _():
        pltpu.sync_copy(
            seg2d.at[pl.ds(_A_SEG_ROWS * _NSUB + _A_SEG_XTRA * s, _A_SEG_XTRA)],
            ids_x)

    def cnt_fire(j, _):
        pltpu.async_copy(ones_v, counts_sh.at[ids_v.at[j]], sem_a, add=True)
        return 0
    lax.fori_loop(0, _A_SEG_ROWS, cnt_fire, 0)

    @pl.when(s < 4)
    def _():
        def cnt_tail(j, _):
            pltpu.async_copy(ones_v, counts_sh.at[ids_x.at[j]], sem_a, add=True)
            return 0
        lax.fori_loop(0, _A_SEG_XTRA, cnt_tail, 0)

    def gc_fire(j, _):
        pltpu.async_copy(ones_v.at[pl.ds(0, 64)], gc_sh.at[gids_v.at[j]],
                         sem_a, add=True)
        return 0
    lax.fori_loop(0, _A_GID_ROWS, gc_fire, 0)

    def cnt_drain(j, _):
        pltpu.make_async_copy(ones_v, counts_sh.at[ids_v.at[0]], sem_a).wait()
        return 0
    lax.fori_loop(0, _A_SEG_ROWS, cnt_drain, 0)

    @pl.when(s < 4)
    def _():
        def tail_drain(j, _):
            pltpu.make_async_copy(ones_v, counts_sh.at[ids_x.at[0]],
                                  sem_a).wait()
            return 0
        lax.fori_loop(0, _A_SEG_XTRA, tail_drain, 0)

    def gc_drain(j, _):
        pltpu.make_async_copy(ones_v.at[pl.ds(0, 64)], gc_sh.at[gids_v.at[0]],
                              sem_a).wait()
        return 0
    lax.fori_loop(0, _A_GID_ROWS, gc_drain, 0)

    plsc.subcore_barrier()

    base_s = (_NS // _NC) * c + _A_SEG_PER_TILE * s
    pltpu.sync_copy(counts_sh.at[pl.ds(base_s, _A_SEG_PER_TILE)], cnt_v)

    def q_body(i, _):
        sl = pl.ds(16 * i, 16)
        qv[sl] = 1.0 / jnp.maximum(cnt_v[sl], 1.0)
        return 0
    lax.fori_loop(0, _A_SEG_PER_TILE // 16, q_body, 0)

    pltpu.sync_copy(qv, q_out.at[pl.ds(base_s, _A_SEG_PER_TILE)])

    @pl.when(jnp.logical_and(c == 0, s == 0))
    def _():
        pltpu.sync_copy(gc_sh, gc_v)
        pltpu.sync_copy(gc_v, gc_out)


@functools.partial(
    pl.kernel,
    out_type=jax.ShapeDtypeStruct((_NW, _NG, _D), jnp.float32),
    mesh=_mesh(),
    scratch_types=[
        pltpu.VMEM((_B_NCHUNK * _B_CHUNK,), jnp.int32),
        pltpu.VMEM((_B_CHUNK,), jnp.float32),
        pltpu.VMEM((_B_CHUNK,), jnp.float32),
        pltpu.VMEM((_B_CHUNK,), jnp.float32),
        pltpu.VMEM((_B_CHUNK,), jnp.float32),
        pltpu.VMEM((_B_CHUNK,), jnp.int32),
        pltpu.VMEM((_B_CHUNK,), jnp.int32),
        pltpu.VMEM((_B_CHUNK,), jnp.int32),
        pltpu.VMEM((_B_CHUNK,), jnp.int32),
        pltpu.VMEM((_B_CHUNK, _D), jnp.float32),
        pltpu.VMEM((_B_CHUNK, _D), jnp.float32),
        pltpu.VMEM((_B_CHUNK, _D), jnp.float32),
        pltpu.VMEM((_B_CHUNK, _D), jnp.float32),
        pltpu.VMEM((_NG, _D), jnp.float32),
        pltpu.SemaphoreType.DMA,
        pltpu.SemaphoreType.DMA,
        pltpu.SemaphoreType.DMA,
        pltpu.SemaphoreType.DMA,
        pltpu.SemaphoreType.DMA,
        pltpu.SemaphoreType.DMA,
        pltpu.SemaphoreType.DMA,
        pltpu.SemaphoreType.DMA,
        pltpu.SemaphoreType.DMA,
        pltpu.SemaphoreType.DMA,
        pltpu.SemaphoreType.DMA,
        pltpu.SemaphoreType.DMA,
    ],
    compiler_params=pltpu.CompilerParams(needs_layout_passes=False),
)
def _pass_b(edge_feat, seg1d, g1d, q_hbm, out, ids_v,
            w_v0, w_v1, w_v2, w_v3, g_v0, g_v1, g_v2, g_v3,
            buf0, buf1, buf2, buf3, acc,
            sem_w0, sem_w1, sem_w2, sem_w3,
            sem_g0, sem_g1, sem_g2, sem_g3,
            sem_x0, sem_x1, sem_x2, sem_x3):
    c = lax.axis_index("c")
    s = lax.axis_index("s")
    wid = s * _NC + c
    n_tile = _B_NCHUNK * _B_CHUNK

    slots = ((w_v0, g_v0, buf0, sem_w0, sem_g0, sem_x0),
             (w_v1, g_v1, buf1, sem_w1, sem_g1, sem_x1),
             (w_v2, g_v2, buf2, sem_w2, sem_g2, sem_x2),
             (w_v3, g_v3, buf3, sem_w3, sem_g3, sem_x3))

    pltpu.sync_copy(seg1d.at[pl.ds(n_tile * wid, n_tile)], ids_v)

    def _start(j, slot):
        w_v, g_v, buf, sem_w, sem_g, sem_x = slot
        idx_row = ids_v.at[pl.ds(_B_CHUNK * j, _B_CHUNK)]
        pltpu.async_copy(q_hbm.at[idx_row], w_v, sem_w)
        pltpu.async_copy(g1d.at[idx_row], g_v, sem_g)
        base_e = _B_CHUNK * (_B_NCHUNK * wid + j)
        pltpu.async_copy(edge_feat.at[pl.ds(base_e, _B_CHUNK)], buf, sem_x)

    def _wait(slot):
        w_v, g_v, buf, sem_w, sem_g, sem_x = slot
        pltpu.make_async_copy(q_hbm.at[pl.ds(0, _B_CHUNK)], w_v, sem_w).wait()
        pltpu.make_async_copy(g1d.at[pl.ds(0, _B_CHUNK)], g_v, sem_g).wait()
        pltpu.make_async_copy(edge_feat.at[pl.ds(0, _B_CHUNK)], buf, sem_x).wait()

    iota16 = lax.iota(jnp.int32, 16)
    _gdn = lax.GatherDimensionNumbers(
        offset_dims=(), collapsed_slice_dims=(0,), start_index_map=(0,))

    def _lane_bcast(vec, lane):
        idx = jnp.full((16, 1), lane, jnp.int32)
        return lax.gather(vec, idx, _gdn, slice_sizes=(1,),
                          mode=lax.GatherScatterMode.PROMISE_IN_BOUNDS)

    def _flush(R, gcur):
        for col in range(_D // 16):
            plsc.addupdate_scatter(acc, [gcur, iota16 + 16 * col], R[col])

    def _accum_group(k, wv, buf, R):
        Rl = list(R)
        for lane in range(16):
            wsp = _lane_bcast(wv, lane)
            i = 16 * k + lane
            for col in range(_D // 16):
                Rl[col] = Rl[col] + buf[i, pl.ds(16 * col, 16)] * wsp
        return tuple(Rl)

    def _compute(slot, carry):
        w_v, g_v, buf, _, _, _ = slot
        R0, gcur0 = carry
        gvs = [g_v[pl.ds(16 * k, 16)] for k in range(_B_CHUNK // 16)]
        u = jnp.all(gvs[0] == gcur0)
        for k in range(1, _B_CHUNK // 16):
            u = jnp.logical_and(u, jnp.all(gvs[k] == gcur0))

        def fast(R, gcur):
            def gbody(k, Rt):
                wv = w_v[pl.ds(16 * k, 16)]
                return _accum_group(k, wv, buf, Rt)
            return lax.fori_loop(0, _B_CHUNK // 16, gbody, R), gcur

        def slow(R, gcur):
            def gbody(k, car):
                Rk, gk = car
                wv = w_v[pl.ds(16 * k, 16)]
                gv = g_v[pl.ds(16 * k, 16)]
                uniform = jnp.all(gv == gk)

                def gfast(Rx, gx):
                    def lane_body(lane, Rt):
                        wsp = _lane_bcast(wv, lane)
                        i = 16 * k + lane
                        return tuple(
                            Rt[col] + buf[i, pl.ds(16 * col, 16)] * wsp
                            for col in range(_D // 16))
                    return lax.fori_loop(0, 16, lane_body, Rx), gx

                def gslow(Rx, gx):
                    _flush(Rx, gx)

                    def lane_body(lane, _):
                        wsp = _lane_bcast(wv, lane)
                        gsp = _lane_bcast(gv, lane)
                        i = 16 * k + lane
                        for col in range(_D // 16):
                            plsc.addupdate_scatter(
                                acc, [gsp, iota16 + 16 * col],
                                buf[i, pl.ds(16 * col, 16)] * wsp)
                        return 0
                    lax.fori_loop(0, 16, lane_body, 0)
                    z = jnp.zeros((16,), jnp.float32)
                    return (tuple(z for _ in range(_D // 16)),
                            _lane_bcast(gv, 15))

                return lax.cond(uniform, gfast, gslow, Rk, gk)
            return lax.fori_loop(0, _B_CHUNK // 16, gbody, (R, gcur))

        return lax.cond(u, fast, slow, R0, gcur0)

    _start(0, slots[0])
    _start(1, slots[1])
    _start(2, slots[2])

    def z_body(r, _):
        for col in range(_D // 16):
            acc[r, pl.ds(16 * col, 16)] = jnp.zeros((16,), jnp.float32)
        return 0
    lax.fori_loop(0, _NG, z_body, 0)

    carry0 = (tuple(jnp.zeros((16,), jnp.float32) for _ in range(_D // 16)),
              jnp.zeros((16,), jnp.int32))

    def quad_body(t, carry):
        j0 = 4 * t
        for u in range(4):
            j = j0 + u

            @pl.when(j + 3 < _B_NCHUNK)
            def _():
                _start(j + 3, slots[(u + 3) % 4])

            def do(R, gcur):
                _wait(slots[u])
                return _compute(slots[u], (R, gcur))

            def skip(R, gcur):
                return (R, gcur)

            carry = lax.cond(j < _B_NCHUNK, do, skip, *carry)
        return carry
    carry = lax.fori_loop(0, (_B_NCHUNK + 3) // 4, quad_body, carry0)

    R, gcur = carry
    _flush(R, gcur)

    pltpu.sync_copy(acc, out.at[wid])


def _sum_body(x_ref, gc_ref, o_ref):
    tot = jnp.sum(x_ref[...], axis=0)
    gc = jnp.maximum(gc_ref[...], 1.0)
    o_ref[...] = tot / gc[:, None]


def kernel(edge_feat, segment_ids, graph_ids):
    seg_i = segment_ids.astype(jnp.int32)
    g_i = graph_ids.astype(jnp.int32)
    seg2d = seg_i.reshape(_NE // 80, 80)
    gids2d = g_i.reshape(_NS // 64, 64)

    q, gc = _pass_a(seg2d, gids2d)
    part = _pass_b(edge_feat, seg_i, g_i, q)

    pooled = pl.pallas_call(
        _sum_body,
        out_shape=jax.ShapeDtypeStruct((_NG, _D), jnp.float32),
    )(part, gc)
    return pooled
